# baseline (speedup 1.0000x reference)
import math

import numpy as np
import jax
import jax.numpy as jnp
from jax import lax
from jax.experimental import pallas as pl
from jax.experimental.pallas import tpu as pltpu
from jax.experimental.pallas import tpu_sc as plsc

VOCAB = 1000000
EMB = 64
MAX_LEN = 512
BATCH = 4096
SEQ = 200
SCALE = math.sqrt(EMB)

NC = 2
NS = 16
L = 16
NW = NC * NS
SEQ_PER_W = BATCH // NW
TOK_PER_W = SEQ_PER_W * SEQ


def _pos_embedding_np():
    rng = np.exp(-np.arange(0, EMB, 2, dtype=np.float64) * math.log(10000) / EMB)
    pos = np.arange(0, MAX_LEN, dtype=np.float64).reshape(MAX_LEN, 1)
    pe = np.zeros((MAX_LEN, EMB), dtype=np.float32)
    pe[:, 0::2] = np.sin(pos * rng).astype(np.float32)
    pe[:, 1::2] = np.cos(pos * rng).astype(np.float32)
    return pe[:SEQ]


_PE = _pos_embedding_np()


GCHUNK = 128


def _sc_body(tokens_hbm, pe_hbm, table_hbm, out_hbm, pe_v, idx_v, row_v,
             gsem):
    wid = lax.axis_index("s") * NC + lax.axis_index("c")
    i0 = wid * SEQ_PER_W

    pltpu.sync_copy(pe_hbm, pe_v)

    def seq_body(i, carry):
        pltpu.sync_copy(tokens_hbm.at[i0 + i], idx_v)
        copies = []
        for c0 in range(0, SEQ, GCHUNK):
            n = min(GCHUNK, SEQ - c0)
            copies.append(pltpu.async_copy(
                table_hbm.at[idx_v.at[pl.ds(c0, n)]],
                row_v.at[pl.ds(c0, n)], gsem))
        for cp in copies:
            cp.wait()

        def r_body(r, c2):
            for qq in range(EMB // L):
                row_v[r, pl.ds(qq * L, L)] = (
                    row_v[r, pl.ds(qq * L, L)] * SCALE
                    + pe_v[r, pl.ds(qq * L, L)])
            return c2

        lax.fori_loop(0, SEQ, r_body, 0, unroll=4)
        pltpu.sync_copy(row_v, out_hbm.at[i0 + i])
        return carry

    lax.fori_loop(0, SEQ_PER_W, seq_body, 0)


def kernel(tokens, embedding_weight):
    tokens_i32 = tokens.astype(jnp.int32)
    pe = jnp.asarray(_PE)
    mesh = plsc.VectorSubcoreMesh(
        core_axis_name="c", subcore_axis_name="s", num_cores=NC,
        num_subcores=NS)
    k = pl.kernel(
        _sc_body,
        out_type=jax.ShapeDtypeStruct((BATCH, SEQ, EMB), jnp.float32),
        mesh=mesh,
        scratch_types=[
            pltpu.VMEM((SEQ, EMB), jnp.float32),
            pltpu.VMEM((SEQ,), jnp.int32),
            pltpu.VMEM((SEQ, EMB), jnp.float32),
            pltpu.SemaphoreType.DMA,
        ],
        compiler_params=pltpu.CompilerParams(use_tc_tiling_on_sc=False),
    )
    return k(tokens_i32, pe, embedding_weight)


# speedup vs baseline: 1.0628x; 1.0628x over previous
import math

import numpy as np
import jax
import jax.numpy as jnp
from jax import lax
from jax.experimental import pallas as pl
from jax.experimental.pallas import tpu as pltpu
from jax.experimental.pallas import tpu_sc as plsc

VOCAB = 1000000
EMB = 64
MAX_LEN = 512
BATCH = 4096
SEQ = 200
SCALE = math.sqrt(EMB)

NC = 2
NS = 16
L = 16
NW = NC * NS
SEQ_PER_W = BATCH // NW

GCHUNK = 128
NROW = 3
NIDX = 2


def _pos_embedding_np():
    rng = np.exp(-np.arange(0, EMB, 2, dtype=np.float64) * math.log(10000) / EMB)
    pos = np.arange(0, MAX_LEN, dtype=np.float64).reshape(MAX_LEN, 1)
    pe = np.zeros((MAX_LEN, EMB), dtype=np.float32)
    pe[:, 0::2] = np.sin(pos * rng).astype(np.float32)
    pe[:, 1::2] = np.cos(pos * rng).astype(np.float32)
    return pe[:SEQ]


_PE = _pos_embedding_np()


def _sc_body(tokens_hbm, pe_hbm, table_hbm, out_hbm, pe_v, idx_v, row_v,
             isem, gsem, ssem):
    wid = lax.axis_index("s") * NC + lax.axis_index("c")
    i0 = wid * SEQ_PER_W

    pltpu.sync_copy(pe_hbm, pe_v)

    def start_idx(i, b):
        pltpu.async_copy(tokens_hbm.at[i0 + i], idx_v.at[b], isem.at[b])

    def wait_idx(i, b):
        pltpu.make_async_copy(tokens_hbm.at[i0 + i], idx_v.at[b],
                              isem.at[b]).wait()

    def start_gather(ib, rb):
        for c0 in range(0, SEQ, GCHUNK):
            n = min(GCHUNK, SEQ - c0)
            pltpu.async_copy(table_hbm.at[idx_v.at[ib, pl.ds(c0, n)]],
                             row_v.at[rb, pl.ds(c0, n)], gsem.at[rb])

    def wait_gather(ib, rb):
        for c0 in range(0, SEQ, GCHUNK):
            n = min(GCHUNK, SEQ - c0)
            pltpu.make_async_copy(table_hbm.at[idx_v.at[ib, pl.ds(c0, n)]],
                                  row_v.at[rb, pl.ds(c0, n)],
                                  gsem.at[rb]).wait()

    def start_store(i, rb):
        pltpu.async_copy(row_v.at[rb], out_hbm.at[i0 + i], ssem.at[rb])

    def wait_store(i, rb):
        pltpu.make_async_copy(row_v.at[rb], out_hbm.at[i0 + i],
                              ssem.at[rb]).wait()

    start_idx(0, 0)
    wait_idx(0, 0)
    start_gather(0, 0)
    start_idx(1, 1)

    def seq_body(i, carry):
        rb = lax.rem(i, NROW)
        rb1 = lax.rem(i + 1, NROW)
        ib = lax.rem(i, NIDX)
        ib1 = lax.rem(i + 1, NIDX)

        @pl.when(i + 1 < SEQ_PER_W)
        def _():
            wait_idx(i + 1, ib1)

            @pl.when(i >= NROW - 1)
            def _():
                wait_store(i - (NROW - 1), rb1)

            start_gather(ib1, rb1)

        wait_gather(ib, rb)

        @pl.when(i + 2 < SEQ_PER_W)
        def _():
            start_idx(i + 2, ib)

        def r_body(r, c2):
            for qq in range(EMB // L):
                row_v[rb, r, pl.ds(qq * L, L)] = (
                    row_v[rb, r, pl.ds(qq * L, L)] * SCALE
                    + pe_v[r, pl.ds(qq * L, L)])
            return c2

        lax.fori_loop(0, SEQ, r_body, 0, unroll=4)
        start_store(i, rb)
        return carry

    lax.fori_loop(0, SEQ_PER_W, seq_body, 0)

    for k in range(NROW):
        i = SEQ_PER_W - NROW + k
        wait_store(i, lax.rem(jnp.int32(i), NROW))


def kernel(tokens, embedding_weight):
    tokens_i32 = tokens.astype(jnp.int32)
    pe = jnp.asarray(_PE)
    mesh = plsc.VectorSubcoreMesh(
        core_axis_name="c", subcore_axis_name="s", num_cores=NC,
        num_subcores=NS)
    k = pl.kernel(
        _sc_body,
        out_type=jax.ShapeDtypeStruct((BATCH, SEQ, EMB), jnp.float32),
        mesh=mesh,
        scratch_types=[
            pltpu.VMEM((SEQ, EMB), jnp.float32),
            pltpu.VMEM((NIDX, SEQ), jnp.int32),
            pltpu.VMEM((NROW, SEQ, EMB), jnp.float32),
            pltpu.SemaphoreType.DMA((NIDX,)),
            pltpu.SemaphoreType.DMA((NROW,)),
            pltpu.SemaphoreType.DMA((NROW,)),
        ],
        compiler_params=pltpu.CompilerParams(use_tc_tiling_on_sc=False),
    )
    return k(tokens_i32, pe, embedding_weight)
